# Initial kernel scaffold; baseline (speedup 1.0000x reference)
import jax
import jax.numpy as jnp
from jax.experimental import pallas as pl


def kernel(x, edge_index, W_l, b_l, W_r, gamma, beta, W_gate, b_gate):
    raise NotImplementedError("write your pallas kernel here")


# speedup vs baseline: 4.0492x; 4.0492x over previous
import functools

import jax
import jax.numpy as jnp
from jax import lax
from jax.experimental import pallas as pl
from jax.experimental.pallas import tpu as pltpu
from jax.experimental.pallas import tpu_sc as plsc

N = 10000
D = 128
E = 320000

DP = 144
NCORE = 2
NSUB = 16
NTILE = NCORE * NSUB
CH = 128
EPT = E // NTILE
NCHUNK = 80
EPT_PAD = NCHUNK * CH
NP = 10112
RPT = NP // NSUB


@functools.cache
def _get_sc_aggregate():
    mesh = plsc.VectorSubcoreMesh(core_axis_name="c", subcore_axis_name="s")

    @functools.partial(
        pl.kernel,
        out_type=jax.ShapeDtypeStruct((NCORE, NP, DP), jnp.float32),
        mesh=mesh,
        scratch_types=[
            pltpu.VMEM((NCHUNK, CH), jnp.int32),
            pltpu.VMEM((NCHUNK, CH), jnp.int32),
            pltpu.VMEM((CH, DP), jnp.float32),
            pltpu.VMEM_SHARED((NP, DP), jnp.float32),
            pltpu.SemaphoreType.DMA,
        ],
        compiler_params=pltpu.CompilerParams(use_tc_tiling_on_sc=False),
    )
    def _sc_aggregate(xpad_hbm, src_hbm, dst_hbm, zero_hbm, out_hbm,
                      src_v, dst_v, rows_v, acc_sh, sem):
        c = lax.axis_index("c")
        s = lax.axis_index("s")
        w = c * NSUB + s
        pltpu.sync_copy(src_hbm.at[w], src_v)
        pltpu.sync_copy(dst_hbm.at[w], dst_v)
        rs = s * RPT
        pltpu.sync_copy(zero_hbm.at[pl.ds(rs, RPT)], acc_sh.at[pl.ds(rs, RPT)])
        plsc.subcore_barrier()

        def body(j, carry):
            pltpu.async_copy(xpad_hbm.at[src_v.at[j]], rows_v, sem).wait()
            pltpu.sync_copy(rows_v, acc_sh.at[dst_v.at[j]], add=True)
            return carry

        lax.fori_loop(0, NCHUNK, body, 0)
        plsc.subcore_barrier()
        pltpu.sync_copy(acc_sh.at[pl.ds(rs, RPT)], out_hbm.at[c, pl.ds(rs, RPT)])

    return _sc_aggregate


def _dense_body(x_ref, acc_ref, wl_ref, bl_ref, wr_ref, g_ref, be_ref,
                wg_ref, bg_ref, o_ref):
    x = x_ref[...]
    a0 = acc_ref[0]
    a1 = acc_ref[1]
    agg = a0[:N, :D] + a1[:N, :D]
    degp = a0[:N, D:DP] + a1[:N, D:DP]
    deg = jnp.sum(degp, axis=1, keepdims=True)
    mean_agg = agg / jnp.maximum(deg, 1.0)
    dn = (((1,), (1,)), ((), ()))
    h = (lax.dot_general(mean_agg, wl_ref[...], dn,
                         preferred_element_type=jnp.float32)
         + bl_ref[...]
         + lax.dot_general(x, wr_ref[...], dn,
                           preferred_element_type=jnp.float32))
    mu = jnp.mean(h, axis=0, keepdims=True)
    var = jnp.mean((h - mu) ** 2, axis=0, keepdims=True)
    hn = (h - mu) * lax.rsqrt(var + 1e-5) * g_ref[...] + be_ref[...]
    hg = 0.5 * hn * (1.0 + lax.erf(hn * 0.7071067811865476))
    wg = wg_ref[...]
    logit = (lax.dot_general(x, wg[:, :D], dn,
                             preferred_element_type=jnp.float32)
             + lax.dot_general(hg, wg[:, D:], dn,
                               preferred_element_type=jnp.float32)
             + bg_ref[...])
    gate = jax.nn.sigmoid(logit)
    o_ref[...] = gate * x + (1.0 - gate) * hg


_dense = pl.pallas_call(
    _dense_body,
    out_shape=jax.ShapeDtypeStruct((N, D), jnp.float32),
)


def kernel(x, edge_index, W_l, b_l, W_r, gamma, beta, W_gate, b_gate):
    src = edge_index[0].astype(jnp.int32)
    dst = edge_index[1].astype(jnp.int32)
    pad = EPT_PAD * NTILE - E
    src_t = jnp.concatenate([src, jnp.zeros((pad,), jnp.int32)]
                            ).reshape(NTILE, NCHUNK, CH)
    dst_t = jnp.concatenate([dst, jnp.full((pad,), N, jnp.int32)]
                            ).reshape(NTILE, NCHUNK, CH)
    x_pad = jnp.concatenate(
        [x, jnp.ones((N, 1), jnp.float32), jnp.zeros((N, DP - D - 1), jnp.float32)],
        axis=1)
    zeros = jnp.zeros((NP, DP), jnp.float32)
    acc = _get_sc_aggregate()(x_pad, src_t, dst_t, zeros)
    return _dense(x, acc, W_l, b_l.reshape(1, D), W_r, gamma.reshape(1, D),
                  beta.reshape(1, D), W_gate, b_gate.reshape(1, D))


# speedup vs baseline: 6.5191x; 1.6100x over previous
import functools

import jax
import jax.numpy as jnp
from jax import lax
from jax.experimental import pallas as pl
from jax.experimental.pallas import tpu as pltpu
from jax.experimental.pallas import tpu_sc as plsc

N = 10000
D = 128
E = 320000

NCORE = 2
NSUB = 16
NTILE = NCORE * NSUB
CH = 64
NCHUNK = 158
EPT_PAD = NCHUNK * CH
NP = 10112
RPT = NP // NSUB
DW = 16


@functools.cache
def _get_sc_aggregate():
    mesh = plsc.VectorSubcoreMesh(core_axis_name="c", subcore_axis_name="s")

    @functools.partial(
        pl.kernel,
        out_type=(jax.ShapeDtypeStruct((NCORE, NP, D), jnp.float32),
                  jax.ShapeDtypeStruct((NCORE, NP, DW), jnp.float32)),
        mesh=mesh,
        scratch_types=[
            pltpu.VMEM((NCHUNK, CH), jnp.int32),
            pltpu.VMEM((NCHUNK, CH), jnp.int32),
            pltpu.VMEM((CH, D), jnp.float32),
            pltpu.VMEM((CH, D), jnp.float32),
            pltpu.VMEM((CH, DW), jnp.float32),
            pltpu.VMEM((CH,), jnp.int32),
            pltpu.VMEM_SHARED((NP, D), jnp.float32),
            pltpu.VMEM_SHARED((NP, DW), jnp.float32),
            pltpu.SemaphoreType.DMA,
            pltpu.SemaphoreType.DMA,
            pltpu.SemaphoreType.DMA,
        ],
        compiler_params=pltpu.CompilerParams(use_tc_tiling_on_sc=False),
    )
    def _sc_aggregate(x_hbm, src_hbm, dst_hbm, zero_hbm, zerod_hbm, ones_hbm,
                      trash_hbm, out_hbm, outd_hbm,
                      src_v, dst_v, rows0_v, rows1_v, ones_v, trash_v,
                      acc_sh, dega_sh, sem0, sem1, semd):
        c = lax.axis_index("c")
        s = lax.axis_index("s")
        w = c * NSUB + s
        pltpu.sync_copy(src_hbm.at[w], src_v)
        pltpu.sync_copy(dst_hbm.at[w], dst_v)
        pltpu.sync_copy(ones_hbm, ones_v)
        pltpu.sync_copy(trash_hbm, trash_v)
        rs = s * RPT
        pltpu.sync_copy(zero_hbm.at[pl.ds(rs, RPT)], acc_sh.at[pl.ds(rs, RPT)])
        pltpu.sync_copy(zerod_hbm.at[pl.ds(rs, RPT)],
                        dega_sh.at[pl.ds(rs, RPT)])
        plsc.subcore_barrier()

        pltpu.async_copy(x_hbm.at[src_v.at[0]], rows0_v, sem0)
        pltpu.async_copy(ones_v, dega_sh.at[trash_v], semd, add=True)
        pltpu.async_copy(ones_v, dega_sh.at[trash_v], semd, add=True)

        def body(i, carry):
            j0 = 2 * i
            j1 = j0 + 1
            pltpu.async_copy(x_hbm.at[src_v.at[j1]], rows1_v, sem1)
            pltpu.async_copy(ones_v, dega_sh.at[dst_v.at[j0]], semd, add=True)
            pltpu.make_async_copy(x_hbm.at[src_v.at[j0]], rows0_v,
                                  sem0).wait()
            pltpu.sync_copy(rows0_v, acc_sh.at[dst_v.at[j0]], add=True)
            j2 = jnp.minimum(j0 + 2, NCHUNK - 1)
            pltpu.async_copy(x_hbm.at[src_v.at[j2]], rows0_v, sem0)
            pltpu.async_copy(ones_v, dega_sh.at[dst_v.at[j1]], semd, add=True)
            pltpu.make_async_copy(ones_v, dega_sh.at[trash_v], semd).wait()
            pltpu.make_async_copy(x_hbm.at[src_v.at[j1]], rows1_v,
                                  sem1).wait()
            pltpu.sync_copy(rows1_v, acc_sh.at[dst_v.at[j1]], add=True)
            pltpu.make_async_copy(ones_v, dega_sh.at[trash_v], semd).wait()
            return carry

        lax.fori_loop(0, NCHUNK // 2, body, 0)
        pltpu.make_async_copy(x_hbm.at[src_v.at[NCHUNK - 1]], rows0_v,
                              sem0).wait()
        pltpu.make_async_copy(ones_v, dega_sh.at[trash_v], semd).wait()
        pltpu.make_async_copy(ones_v, dega_sh.at[trash_v], semd).wait()
        plsc.subcore_barrier()
        pltpu.sync_copy(acc_sh.at[pl.ds(rs, RPT)], out_hbm.at[c, pl.ds(rs, RPT)])
        pltpu.sync_copy(dega_sh.at[pl.ds(rs, RPT)],
                        outd_hbm.at[c, pl.ds(rs, RPT)])

    return _sc_aggregate


def _dense_body(x_ref, acc_ref, dega_ref, wl_ref, bl_ref, wr_ref, g_ref,
                be_ref, wg_ref, bg_ref, o_ref):
    x = x_ref[...]
    a0 = acc_ref[0]
    a1 = acc_ref[1]
    agg = a0[:N, :D] + a1[:N, :D]
    degp = dega_ref[0, :N, :] + dega_ref[1, :N, :]
    deg = jnp.sum(degp, axis=1, keepdims=True)
    mean_agg = agg / jnp.maximum(deg, 1.0)
    dn = (((1,), (1,)), ((), ()))
    h = (lax.dot_general(mean_agg, wl_ref[...], dn,
                         preferred_element_type=jnp.float32)
         + bl_ref[...]
         + lax.dot_general(x, wr_ref[...], dn,
                           preferred_element_type=jnp.float32))
    mu = jnp.mean(h, axis=0, keepdims=True)
    var = jnp.mean((h - mu) ** 2, axis=0, keepdims=True)
    hn = (h - mu) * lax.rsqrt(var + 1e-5) * g_ref[...] + be_ref[...]
    hg = 0.5 * hn * (1.0 + lax.erf(hn * 0.7071067811865476))
    wg = wg_ref[...]
    logit = (lax.dot_general(x, wg[:, :D], dn,
                             preferred_element_type=jnp.float32)
             + lax.dot_general(hg, wg[:, D:], dn,
                               preferred_element_type=jnp.float32)
             + bg_ref[...])
    gate = jax.nn.sigmoid(logit)
    o_ref[...] = gate * x + (1.0 - gate) * hg


_dense = pl.pallas_call(
    _dense_body,
    out_shape=jax.ShapeDtypeStruct((N, D), jnp.float32),
)


def kernel(x, edge_index, W_l, b_l, W_r, gamma, beta, W_gate, b_gate):
    src = edge_index[0].astype(jnp.int32)
    dst = edge_index[1].astype(jnp.int32)
    pad = EPT_PAD * NTILE - E
    src_t = jnp.concatenate([src, jnp.zeros((pad,), jnp.int32)]
                            ).reshape(NTILE, NCHUNK, CH)
    dst_t = jnp.concatenate([dst, jnp.full((pad,), N, jnp.int32)]
                            ).reshape(NTILE, NCHUNK, CH)
    zeros = jnp.zeros((NP, D), jnp.float32)
    zerod = jnp.zeros((NP, DW), jnp.float32)
    ones = jnp.zeros((CH, DW), jnp.float32).at[:, 0].set(1.0)
    trash = jnp.full((CH,), N, jnp.int32)
    acc, degs = _get_sc_aggregate()(x, src_t, dst_t, zeros, zerod, ones,
                                    trash)
    return _dense(x, acc, degs, W_l, b_l.reshape(1, D), W_r,
                  gamma.reshape(1, D), beta.reshape(1, D), W_gate,
                  b_gate.reshape(1, D))
